# Initial kernel scaffold; baseline (speedup 1.0000x reference)
import jax
import jax.numpy as jnp
from jax.experimental import pallas as pl


def kernel(scores):
    raise NotImplementedError("write your pallas kernel here")


# speedup vs baseline: 1.1741x; 1.1741x over previous
import jax
import jax.numpy as jnp
from jax.experimental import pallas as pl

NV = 64
NC = 8
K = 16
MAX_ITER = 20
BISECT_STEPS = 25
BUDGET = 8.0
BT = 1024


def _lpsmap_body(s_ref, o_ref):
    s = s_ref[...]
    b = s.shape[1]

    def outer(_, carry):
        u, p = carry
        u_ext = jnp.concatenate([u, u[:8]], axis=0)
        y = jnp.stack([u_ext[8 * c:8 * c + 16] for c in range(NC)], axis=0)
        y = y + p

        x0 = jnp.clip(y, 0.0, 1.0)
        need = jnp.sum(x0, axis=1, keepdims=True) > BUDGET
        hi0 = jnp.maximum(jnp.max(y, axis=1, keepdims=True), 1e-6)
        lo0 = jnp.zeros_like(hi0)

        def bis(_, lohi):
            lo, hi = lohi
            mid = 0.5 * (lo + hi)
            gt = jnp.sum(jnp.clip(y - mid, 0.0, 1.0), axis=1,
                         keepdims=True) > BUDGET
            return jnp.where(gt, mid, lo), jnp.where(gt, hi, mid)

        lo, hi = jax.lax.fori_loop(0, BISECT_STEPS, bis, (lo0, hi0))
        x1 = jnp.clip(y - 0.5 * (lo + hi), 0.0, 1.0)
        z = jnp.where(need, x1, x0)

        p_new = y - z
        zlo = z[:, 0:8, :]
        zhi = z[:, 8:16, :]
        zhi_roll = jnp.concatenate([zhi[7:8], zhi[:7]], axis=0)
        u_new = ((zlo + zhi_roll) * 0.5).reshape(NV, b)
        return u_new, p_new

    u0 = s
    p0 = jnp.zeros((NC, K, b), jnp.float32)
    u, _ = jax.lax.fori_loop(0, MAX_ITER, outer, (u0, p0))
    o_ref[...] = u


@jax.jit
def kernel(scores):
    st = jnp.asarray(scores, jnp.float32).T
    batch = st.shape[1]
    out = pl.pallas_call(
        _lpsmap_body,
        grid=(batch // BT,),
        in_specs=[pl.BlockSpec((NV, BT), lambda i: (0, i))],
        out_specs=pl.BlockSpec((NV, BT), lambda i: (0, i)),
        out_shape=jax.ShapeDtypeStruct((NV, batch), jnp.float32),
    )(st)
    return out.T


# speedup vs baseline: 1.8020x; 1.5347x over previous
import jax
import jax.numpy as jnp
from jax.experimental import pallas as pl

NV = 64
NC = 8
K = 16
MAX_ITER = 20
BISECT_STEPS = 25
BUDGET = 8.0
BT = 512


def _lpsmap_body(s_ref, o_ref):
    s = s_ref[...]
    b = s.shape[1]

    def outer(_, carry):
        u, p = carry
        u_ext = jnp.concatenate([u, u[:8]], axis=0)
        y = jnp.stack([u_ext[8 * c:8 * c + 16] for c in range(NC)], axis=0)
        y = y + p

        x0 = jnp.clip(y, 0.0, 1.0)
        need = jnp.sum(x0, axis=1, keepdims=True) > BUDGET
        hi0 = jnp.maximum(jnp.max(y, axis=1, keepdims=True), 1e-6)
        lo0 = jnp.zeros_like(hi0)

        lo, hi = lo0, hi0
        for _ in range(BISECT_STEPS):
            mid = 0.5 * (lo + hi)
            gt = jnp.sum(jnp.clip(y - mid, 0.0, 1.0), axis=1,
                         keepdims=True) > BUDGET
            lo, hi = jnp.where(gt, mid, lo), jnp.where(gt, hi, mid)
        x1 = jnp.clip(y - 0.5 * (lo + hi), 0.0, 1.0)
        z = jnp.where(need, x1, x0)

        p_new = y - z
        zlo = z[:, 0:8, :]
        zhi = z[:, 8:16, :]
        zhi_roll = jnp.concatenate([zhi[7:8], zhi[:7]], axis=0)
        u_new = ((zlo + zhi_roll) * 0.5).reshape(NV, b)
        return u_new, p_new

    u0 = s
    p0 = jnp.zeros((NC, K, b), jnp.float32)
    u, _ = jax.lax.fori_loop(0, MAX_ITER, outer, (u0, p0))
    o_ref[...] = u


@jax.jit
def kernel(scores):
    st = jnp.asarray(scores, jnp.float32).T
    batch = st.shape[1]
    out = pl.pallas_call(
        _lpsmap_body,
        grid=(batch // BT,),
        in_specs=[pl.BlockSpec((NV, BT), lambda i: (0, i))],
        out_specs=pl.BlockSpec((NV, BT), lambda i: (0, i)),
        out_shape=jax.ShapeDtypeStruct((NV, batch), jnp.float32),
    )(st)
    return out.T


# speedup vs baseline: 4.1975x; 2.3294x over previous
import jax
import jax.numpy as jnp
from jax.experimental import pallas as pl

NV = 64
NC = 8
K = 16
MAX_ITER = 20
BISECT_STEPS = 25
BUDGET = 8.0
BT = 512


def _lpsmap_body(s_ref, o_ref):
    s = s_ref[...]
    b = s.shape[-1]

    def outer(_, carry):
        u_t, p_t = carry
        u_roll = jnp.roll(u_t, -1, axis=1)
        y = jnp.concatenate([u_t, u_roll], axis=0) + p_t

        x0 = jnp.clip(y, 0.0, 1.0)
        need = jnp.sum(x0, axis=0) > BUDGET
        hi = jnp.maximum(jnp.max(y, axis=0), 1e-6)
        lo = jnp.zeros_like(hi)

        for _ in range(BISECT_STEPS):
            mid = 0.5 * (lo + hi)
            gt = jnp.sum(jnp.clip(y - mid[None], 0.0, 1.0), axis=0) > BUDGET
            lo = jnp.where(gt, mid, lo)
            hi = jnp.where(gt, hi, mid)

        x1 = jnp.clip(y - (0.5 * (lo + hi))[None], 0.0, 1.0)
        z = jnp.where(need[None], x1, x0)

        p_new = y - z
        z_hi = jnp.roll(z[8:], 1, axis=1)
        u_new = (z[:8] + z_hi) * 0.5
        return u_new, p_new

    u0 = s
    p0 = jnp.zeros((K, NC, b), jnp.float32)
    u_t, _ = jax.lax.fori_loop(0, MAX_ITER, outer, (u0, p0))
    o_ref[...] = u_t


@jax.jit
def kernel(scores):
    batch = scores.shape[0]
    st = jnp.transpose(scores.astype(jnp.float32).reshape(batch, NC, 8),
                       (2, 1, 0))
    out = pl.pallas_call(
        _lpsmap_body,
        grid=(batch // BT,),
        in_specs=[pl.BlockSpec((8, NC, BT), lambda i: (0, 0, i))],
        out_specs=pl.BlockSpec((8, NC, BT), lambda i: (0, 0, i)),
        out_shape=jax.ShapeDtypeStruct((8, NC, batch), jnp.float32),
    )(st)
    return jnp.transpose(out, (2, 1, 0)).reshape(batch, NV)
